# baseline (speedup 1.0000x reference)
import jax
import jax.numpy as jnp
from jax import lax
from jax.experimental import pallas as pl
from jax.experimental.pallas import tpu as pltpu

N = 10000
B = 1024
NH = 128
BLK = 128


def _loss_body(rb_blk_ref, rb_ref, dsub_ref, out_ref):
    i = pl.program_id(0)
    gram = lax.dot_general(
        rb_blk_ref[...], rb_ref[...],
        dimension_numbers=(((1,), (1,)), ((), ())),
        preferred_element_type=jnp.float32,
    )
    part = jnp.sum(jnp.abs(gram - dsub_ref[...]))

    @pl.when(i == 0)
    def _():
        out_ref[0, 0] = 0.0

    out_ref[0, 0] += part


def _loss(rb, dsub):
    grid = B // BLK
    return pl.pallas_call(
        _loss_body,
        grid=(grid,),
        in_specs=[
            pl.BlockSpec((BLK, NH), lambda i: (i, 0)),
            pl.BlockSpec((B, NH), lambda i: (0, 0)),
            pl.BlockSpec((BLK, B), lambda i: (i, 0)),
        ],
        out_specs=pl.BlockSpec(memory_space=pltpu.SMEM),
        out_shape=jax.ShapeDtypeStruct((1, 1), jnp.float32),
    )(rb, rb, dsub)


def kernel(data, X, D, edge_index, W1, W2):
    src = edge_index[0]
    dst = edge_index[1]
    h1 = jax.nn.relu(X @ W1)
    agg1 = jax.ops.segment_sum(jnp.take(h1, src, axis=0), dst, num_segments=N) + h1
    h2 = jax.nn.relu(agg1 @ W2)
    rep = jax.ops.segment_sum(jnp.take(h2, src, axis=0), dst, num_segments=N) + h2
    rb = jnp.take(rep, data, axis=0)
    dsub = jnp.take(jnp.take(D, data, axis=0), data, axis=1)
    return _loss(rb, dsub).reshape(1)


# speedup vs baseline: 2.6598x; 2.6598x over previous
import functools

import jax
import jax.numpy as jnp
from jax import lax
from jax.experimental import pallas as pl
from jax.experimental.pallas import tpu as pltpu
from jax.experimental.pallas import tpu_sc as plsc

N = 10000
E = 160000
NH = 128
B = 1024

NC = 2
NS = 16
NW = NC * NS

EC = 128
CHUNKS = E // EC
SEG_ITERS = -(-CHUNKS // NW)
ROWS_PER_SUB = 624
TAIL_ROWS = N - NS * ROWS_PER_SUB
TAIL_BASE = NS * ROWS_PER_SUB

_sc_mesh = functools.partial(
    plsc.VectorSubcoreMesh,
    core_axis_name="c", subcore_axis_name="s",
    num_cores=NC, num_subcores=NS,
)


BM1 = 400


def _mm1_body(x_ref, w_ref, o_ref):
    o_ref[...] = jnp.maximum(
        jnp.dot(x_ref[...], w_ref[...], preferred_element_type=jnp.float32),
        0.0,
    )


def _h1(X, W1):
    return pl.pallas_call(
        _mm1_body,
        grid=(N // BM1,),
        in_specs=[
            pl.BlockSpec((BM1, N), lambda m: (m, 0)),
            pl.BlockSpec((N, NH), lambda m: (0, 0)),
        ],
        out_specs=pl.BlockSpec((BM1, NH), lambda m: (m, 0)),
        out_shape=jax.ShapeDtypeStruct((N, NH), jnp.float32),
    )(X, W1)


def _segsum_body(h_hbm, src_hbm, dst_hbm, zer_hbm, out0, out1,
                 srcv, dstv, rows, acc, sem):
    cid = lax.axis_index("c")
    sid = lax.axis_index("s")
    w = sid * NC + cid

    r0 = sid * ROWS_PER_SUB

    @pl.when(cid == 0)
    def _():
        pltpu.sync_copy(h_hbm.at[pl.ds(r0, ROWS_PER_SUB)],
                        acc.at[pl.ds(r0, ROWS_PER_SUB)])

        @pl.when(sid == NS - 1)
        def _():
            pltpu.sync_copy(h_hbm.at[pl.ds(TAIL_BASE, TAIL_ROWS)],
                            acc.at[pl.ds(TAIL_BASE, TAIL_ROWS)])

    @pl.when(cid != 0)
    def _():
        pltpu.sync_copy(zer_hbm.at[pl.ds(0, ROWS_PER_SUB)],
                        acc.at[pl.ds(r0, ROWS_PER_SUB)])

        @pl.when(sid == NS - 1)
        def _():
            pltpu.sync_copy(zer_hbm.at[pl.ds(0, TAIL_ROWS)],
                            acc.at[pl.ds(TAIL_BASE, TAIL_ROWS)])

    plsc.subcore_barrier()

    def body(i, _):
        chunk = i * NW + w

        @pl.when(chunk < CHUNKS)
        def _():
            base = chunk * EC
            pltpu.sync_copy(src_hbm.at[pl.ds(base, EC)], srcv)
            pltpu.sync_copy(dst_hbm.at[pl.ds(base, EC)], dstv)
            pltpu.async_copy(h_hbm.at[srcv], rows, sem).wait()
            pltpu.sync_copy(rows, acc.at[dstv], add=True)
        return 0

    lax.fori_loop(0, SEG_ITERS, body, 0)
    plsc.subcore_barrier()

    @pl.when(cid == 0)
    def _():
        pltpu.sync_copy(acc.at[pl.ds(r0, ROWS_PER_SUB)],
                        out0.at[pl.ds(r0, ROWS_PER_SUB)])

        @pl.when(sid == NS - 1)
        def _():
            pltpu.sync_copy(acc.at[pl.ds(TAIL_BASE, TAIL_ROWS)],
                            out0.at[pl.ds(TAIL_BASE, TAIL_ROWS)])

    @pl.when(cid != 0)
    def _():
        pltpu.sync_copy(acc.at[pl.ds(r0, ROWS_PER_SUB)],
                        out1.at[pl.ds(r0, ROWS_PER_SUB)])

        @pl.when(sid == NS - 1)
        def _():
            pltpu.sync_copy(acc.at[pl.ds(TAIL_BASE, TAIL_ROWS)],
                            out1.at[pl.ds(TAIL_BASE, TAIL_ROWS)])


def _segsum(h, src, dst, zer):
    k = pl.kernel(
        _segsum_body,
        out_type=(
            jax.ShapeDtypeStruct((N, NH), jnp.float32),
            jax.ShapeDtypeStruct((N, NH), jnp.float32),
        ),
        mesh=_sc_mesh(),
        scratch_types=[
            pltpu.VMEM((EC,), jnp.int32),
            pltpu.VMEM((EC,), jnp.int32),
            pltpu.VMEM((EC, NH), jnp.float32),
            pltpu.VMEM_SHARED((N, NH), jnp.float32),
            pltpu.SemaphoreType.DMA,
        ],
    )
    return k(h, src, dst, zer)


BM2 = 1000


def _mm2_body(a_ref, b_ref, w_ref, o_ref):
    agg = a_ref[...] + b_ref[...]
    o_ref[...] = jnp.maximum(
        jnp.dot(agg, w_ref[...], preferred_element_type=jnp.float32), 0.0)


def _h2(p0, p1, W2):
    return pl.pallas_call(
        _mm2_body,
        grid=(N // BM2,),
        in_specs=[
            pl.BlockSpec((BM2, NH), lambda m: (m, 0)),
            pl.BlockSpec((BM2, NH), lambda m: (m, 0)),
            pl.BlockSpec((NH, NH), lambda m: (0, 0)),
        ],
        out_specs=pl.BlockSpec((BM2, NH), lambda m: (m, 0)),
        out_shape=jax.ShapeDtypeStruct((N, NH), jnp.float32),
    )(p0, p1, W2)


PERW = B * B // NW
CH = 4096
DCH = PERW // CH


def _dsub_body(dflat_hbm, fi_hbm, out, iv, vv, sem):
    cid = lax.axis_index("c")
    sid = lax.axis_index("s")
    w = sid * NC + cid

    def body(k, _):
        off = w * PERW + k * CH
        pltpu.sync_copy(fi_hbm.at[pl.ds(off, CH)], iv)
        pltpu.async_copy(dflat_hbm.at[iv], vv, sem).wait()
        pltpu.sync_copy(vv, out.at[pl.ds(off, CH)])
        return 0

    lax.fori_loop(0, DCH, body, 0)


def _dsub(dflat, fi):
    k = pl.kernel(
        _dsub_body,
        out_type=jax.ShapeDtypeStruct((B * B,), jnp.float32),
        mesh=_sc_mesh(),
        scratch_types=[
            pltpu.VMEM((CH,), jnp.int32),
            pltpu.VMEM((CH,), jnp.float32),
            pltpu.SemaphoreType.DMA,
        ],
    )
    return k(dflat, fi)


RPW = B // NW


def _rbgather_body(q0_hbm, q1_hbm, data_hbm, rb0, rb1, dv, r0, r1, sem):
    cid = lax.axis_index("c")
    sid = lax.axis_index("s")
    w = sid * NC + cid
    base = w * RPW

    pltpu.sync_copy(data_hbm.at[pl.ds(base, RPW)], dv)
    pltpu.async_copy(q0_hbm.at[dv], r0, sem).wait()
    pltpu.async_copy(q1_hbm.at[dv], r1, sem).wait()
    pltpu.sync_copy(r0, rb0.at[pl.ds(base, RPW)])
    pltpu.sync_copy(r1, rb1.at[pl.ds(base, RPW)])


def _rbgather(q0, q1, data):
    k = pl.kernel(
        _rbgather_body,
        out_type=(
            jax.ShapeDtypeStruct((B, NH), jnp.float32),
            jax.ShapeDtypeStruct((B, NH), jnp.float32),
        ),
        mesh=_sc_mesh(),
        scratch_types=[
            pltpu.VMEM((RPW,), jnp.int32),
            pltpu.VMEM((RPW, NH), jnp.float32),
            pltpu.VMEM((RPW, NH), jnp.float32),
            pltpu.SemaphoreType.DMA,
        ],
    )
    return k(q0, q1, data)


BLK = 128


def _loss_body(a_ref, b_ref, fa_ref, fb_ref, ds_ref, out_ref):
    i = pl.program_id(0)
    rb = a_ref[...] + b_ref[...]
    rbf = fa_ref[...] + fb_ref[...]
    gram = lax.dot_general(
        rb, rbf,
        dimension_numbers=(((1,), (1,)), ((), ())),
        preferred_element_type=jnp.float32,
    )
    part = jnp.sum(jnp.abs(gram - ds_ref[...]))

    @pl.when(i == 0)
    def _():
        out_ref[0, 0] = 0.0

    out_ref[0, 0] += part


def _loss(rb0, rb1, dsub):
    return pl.pallas_call(
        _loss_body,
        grid=(B // BLK,),
        in_specs=[
            pl.BlockSpec((BLK, NH), lambda i: (i, 0)),
            pl.BlockSpec((BLK, NH), lambda i: (i, 0)),
            pl.BlockSpec((B, NH), lambda i: (0, 0)),
            pl.BlockSpec((B, NH), lambda i: (0, 0)),
            pl.BlockSpec((BLK, B), lambda i: (i, 0)),
        ],
        out_specs=pl.BlockSpec(memory_space=pltpu.SMEM),
        out_shape=jax.ShapeDtypeStruct((1, 1), jnp.float32),
    )(rb0, rb1, rb0, rb1, dsub)


def kernel(data, X, D, edge_index, W1, W2):
    src = edge_index[0]
    dst = edge_index[1]
    zer = jnp.zeros((ROWS_PER_SUB, NH), jnp.float32)

    fi = (data[:, None] * N + data[None, :]).reshape(-1)
    dsub = _dsub(D.reshape(-1), fi)

    h1 = _h1(X, W1)
    p0, p1 = _segsum(h1, src, dst, zer)
    h2 = _h2(p0, p1, W2)
    q0, q1 = _segsum(h2, src, dst, zer)
    rb0, rb1 = _rbgather(q0, q1, data)
    return _loss(rb0, rb1, dsub.reshape(B, B)).reshape(1)


# speedup vs baseline: 2.7821x; 1.0460x over previous
import functools

import jax
import jax.numpy as jnp
from jax import lax
from jax.experimental import pallas as pl
from jax.experimental.pallas import tpu as pltpu
from jax.experimental.pallas import tpu_sc as plsc

N = 10000
E = 160000
NH = 128
B = 1024

NC = 2
NS = 16
NW = NC * NS

EC = 320
CHUNKS = E // EC
SEG_ITERS = -(-CHUNKS // NW)
ROWS_PER_SUB = 624
TAIL_ROWS = N - NS * ROWS_PER_SUB
TAIL_BASE = NS * ROWS_PER_SUB

_sc_mesh = functools.partial(
    plsc.VectorSubcoreMesh,
    core_axis_name="c", subcore_axis_name="s",
    num_cores=NC, num_subcores=NS,
)


BM1 = 400


def _mm1_body(x_ref, w_ref, o_ref):
    o_ref[...] = jnp.maximum(
        jnp.dot(x_ref[...], w_ref[...], preferred_element_type=jnp.float32),
        0.0,
    )


def _h1(X, W1):
    return pl.pallas_call(
        _mm1_body,
        grid=(N // BM1,),
        in_specs=[
            pl.BlockSpec((BM1, N), lambda m: (m, 0)),
            pl.BlockSpec((N, NH), lambda m: (0, 0)),
        ],
        out_specs=pl.BlockSpec((BM1, NH), lambda m: (m, 0)),
        out_shape=jax.ShapeDtypeStruct((N, NH), jnp.float32),
    )(X, W1)


def _segsum_body(h_hbm, src_hbm, dst_hbm, zer_hbm, out0, out1,
                 srcv, dstv, rows, acc, sem):
    cid = lax.axis_index("c")
    sid = lax.axis_index("s")
    w = sid * NC + cid

    r0 = sid * ROWS_PER_SUB

    @pl.when(cid == 0)
    def _():
        pltpu.sync_copy(h_hbm.at[pl.ds(r0, ROWS_PER_SUB)],
                        acc.at[pl.ds(r0, ROWS_PER_SUB)])

        @pl.when(sid == NS - 1)
        def _():
            pltpu.sync_copy(h_hbm.at[pl.ds(TAIL_BASE, TAIL_ROWS)],
                            acc.at[pl.ds(TAIL_BASE, TAIL_ROWS)])

    @pl.when(cid != 0)
    def _():
        pltpu.sync_copy(zer_hbm.at[pl.ds(0, ROWS_PER_SUB)],
                        acc.at[pl.ds(r0, ROWS_PER_SUB)])

        @pl.when(sid == NS - 1)
        def _():
            pltpu.sync_copy(zer_hbm.at[pl.ds(0, TAIL_ROWS)],
                            acc.at[pl.ds(TAIL_BASE, TAIL_ROWS)])

    plsc.subcore_barrier()

    def body(i, _):
        chunk = i * NW + w

        @pl.when(chunk < CHUNKS)
        def _():
            base = chunk * EC
            pltpu.sync_copy(src_hbm.at[pl.ds(base, EC)], srcv)
            pltpu.sync_copy(dst_hbm.at[pl.ds(base, EC)], dstv)
            pltpu.async_copy(h_hbm.at[srcv], rows, sem).wait()
            pltpu.sync_copy(rows, acc.at[dstv], add=True)
        return 0

    lax.fori_loop(0, SEG_ITERS, body, 0)
    plsc.subcore_barrier()

    @pl.when(cid == 0)
    def _():
        pltpu.sync_copy(acc.at[pl.ds(r0, ROWS_PER_SUB)],
                        out0.at[pl.ds(r0, ROWS_PER_SUB)])

        @pl.when(sid == NS - 1)
        def _():
            pltpu.sync_copy(acc.at[pl.ds(TAIL_BASE, TAIL_ROWS)],
                            out0.at[pl.ds(TAIL_BASE, TAIL_ROWS)])

    @pl.when(cid != 0)
    def _():
        pltpu.sync_copy(acc.at[pl.ds(r0, ROWS_PER_SUB)],
                        out1.at[pl.ds(r0, ROWS_PER_SUB)])

        @pl.when(sid == NS - 1)
        def _():
            pltpu.sync_copy(acc.at[pl.ds(TAIL_BASE, TAIL_ROWS)],
                            out1.at[pl.ds(TAIL_BASE, TAIL_ROWS)])


def _segsum(h, src, dst, zer):
    k = pl.kernel(
        _segsum_body,
        out_type=(
            jax.ShapeDtypeStruct((N, NH), jnp.float32),
            jax.ShapeDtypeStruct((N, NH), jnp.float32),
        ),
        mesh=_sc_mesh(),
        scratch_types=[
            pltpu.VMEM((EC,), jnp.int32),
            pltpu.VMEM((EC,), jnp.int32),
            pltpu.VMEM((EC, NH), jnp.float32),
            pltpu.VMEM_SHARED((N, NH), jnp.float32),
            pltpu.SemaphoreType.DMA,
        ],
    )
    return k(h, src, dst, zer)


BM2 = 1000


def _mm2_body(a_ref, b_ref, w_ref, o_ref):
    agg = a_ref[...] + b_ref[...]
    o_ref[...] = jnp.maximum(
        jnp.dot(agg, w_ref[...], preferred_element_type=jnp.float32), 0.0)


def _h2(p0, p1, W2):
    return pl.pallas_call(
        _mm2_body,
        grid=(N // BM2,),
        in_specs=[
            pl.BlockSpec((BM2, NH), lambda m: (m, 0)),
            pl.BlockSpec((BM2, NH), lambda m: (m, 0)),
            pl.BlockSpec((NH, NH), lambda m: (0, 0)),
        ],
        out_specs=pl.BlockSpec((BM2, NH), lambda m: (m, 0)),
        out_shape=jax.ShapeDtypeStruct((N, NH), jnp.float32),
    )(p0, p1, W2)


PERW = B * B // NW
CH = 4096
DCH = PERW // CH


def _dsub_body(dflat_hbm, fi_hbm, out, iv, vv, sem):
    cid = lax.axis_index("c")
    sid = lax.axis_index("s")
    w = sid * NC + cid

    def body(k, _):
        off = w * PERW + k * CH
        pltpu.sync_copy(fi_hbm.at[pl.ds(off, CH)], iv)
        pltpu.async_copy(dflat_hbm.at[iv], vv, sem).wait()
        pltpu.sync_copy(vv, out.at[pl.ds(off, CH)])
        return 0

    lax.fori_loop(0, DCH, body, 0)


def _dsub(dflat, fi):
    k = pl.kernel(
        _dsub_body,
        out_type=jax.ShapeDtypeStruct((B * B,), jnp.float32),
        mesh=_sc_mesh(),
        scratch_types=[
            pltpu.VMEM((CH,), jnp.int32),
            pltpu.VMEM((CH,), jnp.float32),
            pltpu.SemaphoreType.DMA,
        ],
    )
    return k(dflat, fi)


RPW = B // NW


def _rbgather_body(q0_hbm, q1_hbm, data_hbm, rb0, rb1, dv, r0, r1, sem):
    cid = lax.axis_index("c")
    sid = lax.axis_index("s")
    w = sid * NC + cid
    base = w * RPW

    pltpu.sync_copy(data_hbm.at[pl.ds(base, RPW)], dv)
    pltpu.async_copy(q0_hbm.at[dv], r0, sem).wait()
    pltpu.async_copy(q1_hbm.at[dv], r1, sem).wait()
    pltpu.sync_copy(r0, rb0.at[pl.ds(base, RPW)])
    pltpu.sync_copy(r1, rb1.at[pl.ds(base, RPW)])


def _rbgather(q0, q1, data):
    k = pl.kernel(
        _rbgather_body,
        out_type=(
            jax.ShapeDtypeStruct((B, NH), jnp.float32),
            jax.ShapeDtypeStruct((B, NH), jnp.float32),
        ),
        mesh=_sc_mesh(),
        scratch_types=[
            pltpu.VMEM((RPW,), jnp.int32),
            pltpu.VMEM((RPW, NH), jnp.float32),
            pltpu.VMEM((RPW, NH), jnp.float32),
            pltpu.SemaphoreType.DMA,
        ],
    )
    return k(q0, q1, data)


BLK = 128


def _loss_body(a_ref, b_ref, fa_ref, fb_ref, ds_ref, out_ref):
    i = pl.program_id(0)
    rb = a_ref[...] + b_ref[...]
    rbf = fa_ref[...] + fb_ref[...]
    gram = lax.dot_general(
        rb, rbf,
        dimension_numbers=(((1,), (1,)), ((), ())),
        preferred_element_type=jnp.float32,
    )
    part = jnp.sum(jnp.abs(gram - ds_ref[...]))

    @pl.when(i == 0)
    def _():
        out_ref[0, 0] = 0.0

    out_ref[0, 0] += part


def _loss(rb0, rb1, dsub):
    return pl.pallas_call(
        _loss_body,
        grid=(B // BLK,),
        in_specs=[
            pl.BlockSpec((BLK, NH), lambda i: (i, 0)),
            pl.BlockSpec((BLK, NH), lambda i: (i, 0)),
            pl.BlockSpec((B, NH), lambda i: (0, 0)),
            pl.BlockSpec((B, NH), lambda i: (0, 0)),
            pl.BlockSpec((BLK, B), lambda i: (i, 0)),
        ],
        out_specs=pl.BlockSpec(memory_space=pltpu.SMEM),
        out_shape=jax.ShapeDtypeStruct((1, 1), jnp.float32),
    )(rb0, rb1, rb0, rb1, dsub)


def kernel(data, X, D, edge_index, W1, W2):
    src = edge_index[0]
    dst = edge_index[1]
    zer = jnp.zeros((ROWS_PER_SUB, NH), jnp.float32)

    fi = (data[:, None] * N + data[None, :]).reshape(-1)
    dsub = _dsub(D.reshape(-1), fi)

    h1 = _h1(X, W1)
    p0, p1 = _segsum(h1, src, dst, zer)
    h2 = _h2(p0, p1, W2)
    q0, q1 = _segsum(h2, src, dst, zer)
    rb0, rb1 = _rbgather(q0, q1, data)
    return _loss(rb0, rb1, dsub.reshape(B, B)).reshape(1)
